# baseline (speedup 1.0000x reference)
import functools

import jax
import jax.numpy as jnp
from jax import lax
from jax.experimental import pallas as pl
from jax.experimental.pallas import tpu as pltpu

_NUM_CLASSES = 80
_NUM_INSTANCE = 32
_MEM_DIM = 256
_ROI_SIZE = 7
_N = 512
_ND = _NUM_CLASSES * _NUM_INSTANCE
_D = _MEM_DIM * _ROI_SIZE * _ROI_SIZE

_INTERPRET = False


def _prep_body(pcls_r_ref, pcls_c_ref, rcls_r_ref, rcls_c_ref,
               pfeat_ref, ptab_ref, rtab_ref,
               pf_ref, ps_ref, rs_ref, zrow_ref,
               inv_ref, vsrc_ref, vdst_ref, rsrc_ref):
    i32 = jnp.int32
    ii = lax.broadcasted_iota(i32, (_N, _N), 0)
    jj = lax.broadcasted_iota(i32, (_N, _N), 1)
    d_r = lax.broadcasted_iota(i32, (_ND, 1), 0)
    jD = lax.broadcasted_iota(i32, (_ND, _N), 1)

    def side(cls_r, cls_c):
        eq = cls_r == cls_c
        rank_r = jnp.sum((eq & (jj < ii)).astype(i32), axis=1, keepdims=True)
        rank_c = jnp.sum((eq & (ii < jj)).astype(i32), axis=0, keepdims=True)
        dest_r = (cls_r - 1) * _NUM_INSTANCE + (rank_r & (_NUM_INSTANCE - 1))
        dest_c = (cls_c - 1) * _NUM_INSTANCE + (rank_c & (_NUM_INSTANCE - 1))
        samedest = dest_r == dest_c
        wsrc_r = jnp.max(jnp.where(samedest, jj, -1), axis=1, keepdims=True)
        dmat = d_r == dest_c
        srctab_r = jnp.max(jnp.where(dmat, jD, -1), axis=1, keepdims=True)
        onehot = (srctab_r == jD).astype(jnp.float32)
        return dest_r, wsrc_r, srctab_r, onehot

    _, _, _, onehot_p = side(pcls_r_ref[...], pcls_c_ref[...])
    dest_r, wsrc_r, srctab_r, onehot_r = side(rcls_r_ref[...], rcls_c_ref[...])

    pf_ref[...] = jnp.dot(onehot_p, pfeat_ref[...],
                          preferred_element_type=jnp.float32)
    ps_ref[...] = jnp.dot(onehot_p, ptab_ref[...],
                          preferred_element_type=jnp.float32)
    rs_ref[...] = jnp.dot(onehot_r, rtab_ref[...],
                          preferred_element_type=jnp.float32)
    zrow_ref[...] = jnp.zeros_like(zrow_ref)

    valid = srctab_r >= 0
    j0 = jnp.min(jnp.where(valid, _ND, d_r), axis=0, keepdims=True)
    inv_ref[...] = jnp.where(valid, j0, d_r)
    vsrc_ref[...] = wsrc_r
    vdst_ref[...] = dest_r
    rsrc_ref[...] = srctab_r


def _prep(pcls, rcls, pfeat, ptab, rtab):
    i32 = jnp.int32
    f32 = jnp.float32
    out_shapes = (
        jax.ShapeDtypeStruct((_ND, _MEM_DIM), f32),
        jax.ShapeDtypeStruct((_ND, 8), f32),
        jax.ShapeDtypeStruct((_ND, 8), f32),
        jax.ShapeDtypeStruct((4, _D), f32),
        jax.ShapeDtypeStruct((_ND, 1), i32),
        jax.ShapeDtypeStruct((_N, 1), i32),
        jax.ShapeDtypeStruct((_N, 1), i32),
        jax.ShapeDtypeStruct((_ND, 1), i32),
    )
    pcls_r = pcls.reshape(_N, 1)
    pcls_c = pcls.reshape(1, _N)
    rcls_r = rcls.reshape(_N, 1)
    rcls_c = rcls.reshape(1, _N)
    return pl.pallas_call(
        _prep_body,
        out_shape=out_shapes,
        interpret=_INTERPRET,
    )(pcls_r, pcls_c, rcls_r, rcls_c, pfeat, ptab, rtab)


def _tc_gather_body(rsrc_ref, in_ref, out_ref):
    d = pl.program_id(0)
    s = rsrc_ref[d]

    @pl.when(s >= 0)
    def _():
        out_ref[...] = in_ref[...]

    @pl.when(s < 0)
    def _():
        out_ref[...] = jnp.zeros_like(out_ref)


def _tc_gather(rsrc, roi_flat):
    grid_spec = pltpu.PrefetchScalarGridSpec(
        num_scalar_prefetch=1,
        grid=(_ND,),
        in_specs=[
            pl.BlockSpec((1, 1, _D),
                         lambda d, rs: (jnp.maximum(rs[d], 0), 0, 0)),
        ],
        out_specs=pl.BlockSpec((1, 1, _D), lambda d, rs: (d, 0, 0)),
    )
    out = pl.pallas_call(
        _tc_gather_body,
        grid_spec=grid_spec,
        out_shape=jax.ShapeDtypeStruct((_ND, 1, _D), jnp.float32),
        interpret=_INTERPRET,
    )(rsrc, roi_flat.reshape(_N, 1, _D))
    return out.reshape(_ND, _D)


def kernel(prop_class, prop_feature, prop_deltas, prop_scale,
           roi_class, roi_feature, roi_deltas, roi_scale):
    f32 = jnp.float32
    ptab = jnp.concatenate(
        [prop_deltas, prop_scale[:, None],
         jnp.zeros((_N, 3), f32)], axis=1)
    rtab = jnp.concatenate(
        [roi_deltas, roi_scale[:, None],
         jnp.zeros((_N, 3), f32)], axis=1)

    (pf, ps, rs, _zrow, _inv, _vsrc, _vdst, rsrc) = _prep(
        prop_class, roi_class, prop_feature, ptab, rtab)

    roi_flat = roi_feature.reshape(_N, _D)
    roi_mem = _tc_gather(rsrc.reshape(_ND), roi_flat)

    return (
        pf.reshape(_NUM_CLASSES, _NUM_INSTANCE, _MEM_DIM),
        ps[:, :4].reshape(_NUM_CLASSES, _NUM_INSTANCE, 4),
        ps[:, 4].reshape(_NUM_CLASSES, _NUM_INSTANCE),
        roi_mem.reshape(_NUM_CLASSES, _NUM_INSTANCE, _MEM_DIM,
                        _ROI_SIZE, _ROI_SIZE),
        rs[:, :4].reshape(_NUM_CLASSES, _NUM_INSTANCE, 4),
        rs[:, 4].reshape(_NUM_CLASSES, _NUM_INSTANCE),
    )


# speedup vs baseline: 2.2072x; 2.2072x over previous
import functools

import jax
import jax.numpy as jnp
from jax import lax
from jax.experimental import pallas as pl
from jax.experimental.pallas import tpu as pltpu
from jax.experimental.pallas import tpu_sc as plsc

_NUM_CLASSES = 80
_NUM_INSTANCE = 32
_MEM_DIM = 256
_ROI_SIZE = 7
_N = 512
_ND = _NUM_CLASSES * _NUM_INSTANCE
_D = _MEM_DIM * _ROI_SIZE * _ROI_SIZE

_INTERPRET = False


def _prep_body(pcls_r_ref, pcls_c_ref, rcls_r_ref, rcls_c_ref,
               pfeat_ref, ptab_ref, rtab_ref,
               pf_ref, ps_ref, rs_ref, zrow_ref,
               inv_ref, vsrc_ref, vdst_ref, rsrc_ref):
    i32 = jnp.int32
    ii = lax.broadcasted_iota(i32, (_N, _N), 0)
    jj = lax.broadcasted_iota(i32, (_N, _N), 1)
    d_r = lax.broadcasted_iota(i32, (_ND, 1), 0)
    jD = lax.broadcasted_iota(i32, (_ND, _N), 1)

    def side(cls_r, cls_c):
        eq = cls_r == cls_c
        rank_r = jnp.sum((eq & (jj < ii)).astype(i32), axis=1, keepdims=True)
        rank_c = jnp.sum((eq & (ii < jj)).astype(i32), axis=0, keepdims=True)
        dest_r = (cls_r - 1) * _NUM_INSTANCE + (rank_r & (_NUM_INSTANCE - 1))
        dest_c = (cls_c - 1) * _NUM_INSTANCE + (rank_c & (_NUM_INSTANCE - 1))
        samedest = dest_r == dest_c
        wsrc_r = jnp.max(jnp.where(samedest, jj, -1), axis=1, keepdims=True)
        dmat = d_r == dest_c
        srctab_r = jnp.max(jnp.where(dmat, jD, -1), axis=1, keepdims=True)
        onehot = (srctab_r == jD).astype(jnp.float32)
        return dest_r, wsrc_r, srctab_r, onehot

    _, _, _, onehot_p = side(pcls_r_ref[...], pcls_c_ref[...])
    dest_r, wsrc_r, srctab_r, onehot_r = side(rcls_r_ref[...], rcls_c_ref[...])

    pf_ref[...] = jnp.dot(onehot_p, pfeat_ref[...],
                          preferred_element_type=jnp.float32)
    ps_ref[...] = jnp.dot(onehot_p, ptab_ref[...],
                          preferred_element_type=jnp.float32)
    rs_ref[...] = jnp.dot(onehot_r, rtab_ref[...],
                          preferred_element_type=jnp.float32)
    zrow_ref[...] = jnp.zeros_like(zrow_ref)

    valid = srctab_r >= 0
    j0 = jnp.min(jnp.where(valid, _ND, d_r), axis=0, keepdims=True)
    inv_ref[...] = jnp.where(valid, j0, d_r)
    vsrc_ref[...] = wsrc_r
    vdst_ref[...] = dest_r
    rsrc_ref[...] = srctab_r


def _prep(pcls, rcls, pfeat, ptab, rtab):
    i32 = jnp.int32
    f32 = jnp.float32
    out_shapes = (
        jax.ShapeDtypeStruct((_ND, _MEM_DIM), f32),
        jax.ShapeDtypeStruct((_ND, 8), f32),
        jax.ShapeDtypeStruct((_ND, 8), f32),
        jax.ShapeDtypeStruct((4, _D), f32),
        jax.ShapeDtypeStruct((_ND, 1), i32),
        jax.ShapeDtypeStruct((_N, 1), i32),
        jax.ShapeDtypeStruct((_N, 1), i32),
        jax.ShapeDtypeStruct((_ND, 1), i32),
    )
    pcls_r = pcls.reshape(_N, 1)
    pcls_c = pcls.reshape(1, _N)
    rcls_r = rcls.reshape(_N, 1)
    rcls_c = rcls.reshape(1, _N)
    return pl.pallas_call(
        _prep_body,
        out_shape=out_shapes,
        interpret=_INTERPRET,
    )(pcls_r, pcls_c, rcls_r, rcls_c, pfeat, ptab, rtab)


def _tc_gather_body(rsrc_ref, in_ref, out_ref):
    d = pl.program_id(0)
    s = rsrc_ref[d]

    @pl.when(s >= 0)
    def _():
        out_ref[...] = in_ref[...]

    @pl.when(s < 0)
    def _():
        out_ref[...] = jnp.zeros_like(out_ref)


def _tc_gather(rsrc, roi_flat):
    grid_spec = pltpu.PrefetchScalarGridSpec(
        num_scalar_prefetch=1,
        grid=(_ND,),
        in_specs=[
            pl.BlockSpec((1, 1, _D),
                         lambda d, rs: (jnp.maximum(rs[d], 0), 0, 0)),
        ],
        out_specs=pl.BlockSpec((1, 1, _D), lambda d, rs: (d, 0, 0)),
    )
    out = pl.pallas_call(
        _tc_gather_body,
        grid_spec=grid_spec,
        out_shape=jax.ShapeDtypeStruct((_ND, 1, _D), jnp.float32),
        interpret=_INTERPRET,
    )(rsrc, roi_flat.reshape(_N, 1, _D))
    return out.reshape(_ND, _D)


_NC = 2
_NS = 16
_NW = _NC * _NS
_IB = _ND // _NW
_ZB = 4
_VB = 2
_VI = _N // _NW
_ZROUNDS = _IB // _ZB
_VROUNDS = _VI // _VB


def _sc_gather_body(roi_hbm, zrow_hbm, inv_hbm, vsrc_hbm, vdst_hbm, out_hbm,
                    inv_v, vsrc_v, vdst_v, zbuf, gbuf0, gbuf1,
                    zsem, gsem0, gsem1, ssem0, ssem1):
    wid = lax.axis_index("s") * _NC + lax.axis_index("c")
    pltpu.sync_copy(inv_hbm.at[wid], inv_v)
    pltpu.sync_copy(vsrc_hbm.at[wid], vsrc_v)
    pltpu.sync_copy(vdst_hbm.at[wid], vdst_v)
    pltpu.sync_copy(zrow_hbm, zbuf)

    zcopies = [
        pltpu.async_copy(zbuf, out_hbm.at[inv_v.at[j]], zsem)
        for j in range(_ZROUNDS)
    ]

    gbufs = (gbuf0, gbuf1)
    gsems = (gsem0, gsem1)
    ssems = (ssem0, ssem1)
    gh = [None] * _VROUNDS
    sh = [None] * _VROUNDS
    gh[0] = pltpu.async_copy(roi_hbm.at[vsrc_v.at[0]], gbufs[0], gsems[0])
    for j in range(_VROUNDS):
        cur = j & 1
        nxt = cur ^ 1
        gh[j].wait()
        sh[j] = pltpu.async_copy(gbufs[cur], out_hbm.at[vdst_v.at[j]],
                                 ssems[cur])
        if j + 1 < _VROUNDS:
            if j >= 1:
                sh[j - 1].wait()
            gh[j + 1] = pltpu.async_copy(roi_hbm.at[vsrc_v.at[j + 1]],
                                         gbufs[nxt], gsems[nxt])
    sh[_VROUNDS - 1].wait()
    for h in zcopies:
        h.wait()


def _sc_gather(roi_flat, zrow, inv3, vsrc3, vdst3):
    mesh = plsc.VectorSubcoreMesh(core_axis_name="c", subcore_axis_name="s")
    run = pl.kernel(
        _sc_gather_body,
        out_type=jax.ShapeDtypeStruct((_ND, _D), jnp.float32),
        mesh=mesh,
        scratch_types=[
            pltpu.VMEM((_ZROUNDS, _ZB), jnp.int32),
            pltpu.VMEM((_VROUNDS, _VB), jnp.int32),
            pltpu.VMEM((_VROUNDS, _VB), jnp.int32),
            pltpu.VMEM((_ZB, _D), jnp.float32),
            pltpu.VMEM((_VB, _D), jnp.float32),
            pltpu.VMEM((_VB, _D), jnp.float32),
            pltpu.SemaphoreType.DMA,
            pltpu.SemaphoreType.DMA,
            pltpu.SemaphoreType.DMA,
            pltpu.SemaphoreType.DMA,
            pltpu.SemaphoreType.DMA,
        ],
        interpret=_INTERPRET,
    )
    return run(roi_flat, zrow, inv3, vsrc3, vdst3)


def kernel(prop_class, prop_feature, prop_deltas, prop_scale,
           roi_class, roi_feature, roi_deltas, roi_scale):
    f32 = jnp.float32
    ptab = jnp.concatenate(
        [prop_deltas, prop_scale[:, None],
         jnp.zeros((_N, 3), f32)], axis=1)
    rtab = jnp.concatenate(
        [roi_deltas, roi_scale[:, None],
         jnp.zeros((_N, 3), f32)], axis=1)

    (pf, ps, rs, zrow, inv, vsrc, vdst, _rsrc) = _prep(
        prop_class, roi_class, prop_feature, ptab, rtab)

    roi_flat = roi_feature.reshape(_N, _D)
    roi_mem = _sc_gather(roi_flat, zrow,
                         inv.reshape(_NW, _ZROUNDS, _ZB),
                         vsrc.reshape(_NW, _VROUNDS, _VB),
                         vdst.reshape(_NW, _VROUNDS, _VB))

    return (
        pf.reshape(_NUM_CLASSES, _NUM_INSTANCE, _MEM_DIM),
        ps[:, :4].reshape(_NUM_CLASSES, _NUM_INSTANCE, 4),
        ps[:, 4].reshape(_NUM_CLASSES, _NUM_INSTANCE),
        roi_mem.reshape(_NUM_CLASSES, _NUM_INSTANCE, _MEM_DIM,
                        _ROI_SIZE, _ROI_SIZE),
        rs[:, :4].reshape(_NUM_CLASSES, _NUM_INSTANCE, 4),
        rs[:, 4].reshape(_NUM_CLASSES, _NUM_INSTANCE),
    )


# speedup vs baseline: 12.4997x; 5.6632x over previous
import functools

import jax
import jax.numpy as jnp
from jax import lax
from jax.experimental import pallas as pl
from jax.experimental.pallas import tpu as pltpu
from jax.experimental.pallas import tpu_sc as plsc

_NUM_CLASSES = 80
_NUM_INSTANCE = 32
_MEM_DIM = 256
_ROI_SIZE = 7
_N = 512
_ND = _NUM_CLASSES * _NUM_INSTANCE
_HW = _ROI_SIZE * _ROI_SIZE

_NC = 2
_NS = 16
_NW = _NC * _NS
_CB = 112
_ZROUNDS = (_ND * _HW) // (_NW * _CB)
_VROUNDS = (_N * _HW) // (_NW * _CB)

_INTERPRET = False


def _prep_body(pcls_r_ref, pcls_c_ref, rcls_r_ref, rcls_c_ref,
               pfeat_ref, ptab_ref, rtab_ref,
               pf_ref, ps_ref, rs_ref, zrow_ref,
               inv_ref, vsrc_ref, vdst_ref):
    i32 = jnp.int32
    ii = lax.broadcasted_iota(i32, (_N, _N), 0)
    jj = lax.broadcasted_iota(i32, (_N, _N), 1)
    d_r = lax.broadcasted_iota(i32, (_ND, 1), 0)
    jD = lax.broadcasted_iota(i32, (_ND, _N), 1)

    def side(cls_r, cls_c):
        eq = cls_r == cls_c
        rank_r = jnp.sum((eq & (jj < ii)).astype(i32), axis=1, keepdims=True)
        rank_c = jnp.sum((eq & (ii < jj)).astype(i32), axis=0, keepdims=True)
        dest_r = (cls_r - 1) * _NUM_INSTANCE + (rank_r & (_NUM_INSTANCE - 1))
        dest_c = (cls_c - 1) * _NUM_INSTANCE + (rank_c & (_NUM_INSTANCE - 1))
        samedest = dest_r == dest_c
        wsrc_r = jnp.max(jnp.where(samedest, jj, -1), axis=1, keepdims=True)
        dmat = d_r == dest_c
        srctab_r = jnp.max(jnp.where(dmat, jD, -1), axis=1, keepdims=True)
        onehot = (srctab_r == jD).astype(jnp.float32)
        return dest_r, wsrc_r, srctab_r, onehot

    _, _, _, onehot_p = side(pcls_r_ref[...], pcls_c_ref[...])
    dest_r, wsrc_r, srctab_r, onehot_r = side(rcls_r_ref[...], rcls_c_ref[...])

    pf_ref[...] = jnp.dot(onehot_p, pfeat_ref[...],
                          preferred_element_type=jnp.float32)
    ps_ref[...] = jnp.dot(onehot_p, ptab_ref[...],
                          preferred_element_type=jnp.float32)
    rs_ref[...] = jnp.dot(onehot_r, rtab_ref[...],
                          preferred_element_type=jnp.float32)
    zrow_ref[...] = jnp.zeros_like(zrow_ref)

    valid = srctab_r >= 0
    j0 = jnp.min(jnp.where(valid, _ND, d_r), axis=0, keepdims=True)
    hw_d = lax.broadcasted_iota(i32, (_ND, _HW), 1)
    hw_n = lax.broadcasted_iota(i32, (_N, _HW), 1)

    def chunk(d, hw):
        return (d >> 5) * (_HW * _NUM_INSTANCE) + hw * _NUM_INSTANCE + (d & 31)

    dd = jnp.where(valid, j0, d_r)
    inv_ref[...] = chunk(dd, hw_d)
    vsrc_ref[...] = hw_n * _N + wsrc_r
    vdst_ref[...] = chunk(dest_r, hw_n)


def _prep(pcls, rcls, pfeat, ptab, rtab):
    i32 = jnp.int32
    f32 = jnp.float32
    out_shapes = (
        jax.ShapeDtypeStruct((_ND, _MEM_DIM), f32),
        jax.ShapeDtypeStruct((_ND, 8), f32),
        jax.ShapeDtypeStruct((_ND, 8), f32),
        jax.ShapeDtypeStruct((_CB, _MEM_DIM), f32),
        jax.ShapeDtypeStruct((_ND, _HW), i32),
        jax.ShapeDtypeStruct((_N, _HW), i32),
        jax.ShapeDtypeStruct((_N, _HW), i32),
    )
    pcls_r = pcls.reshape(_N, 1)
    pcls_c = pcls.reshape(1, _N)
    rcls_r = rcls.reshape(_N, 1)
    rcls_c = rcls.reshape(1, _N)
    return pl.pallas_call(
        _prep_body,
        out_shape=out_shapes,
        interpret=_INTERPRET,
    )(pcls_r, pcls_c, rcls_r, rcls_c, pfeat, ptab, rtab)


def _sc_gather_body(roi_hbm, zrow_hbm, inv_hbm, vsrc_hbm, vdst_hbm, out_hbm,
                    inv_v, vsrc_v, vdst_v, zbuf, gbuf0, gbuf1,
                    zsem, gsem0, gsem1, ssem0, ssem1):
    wid = lax.axis_index("s") * _NC + lax.axis_index("c")
    pltpu.sync_copy(inv_hbm.at[wid], inv_v)
    pltpu.sync_copy(vsrc_hbm.at[wid], vsrc_v)
    pltpu.sync_copy(vdst_hbm.at[wid], vdst_v)
    pltpu.sync_copy(zrow_hbm, zbuf)

    zcopies = [
        pltpu.async_copy(zbuf, out_hbm.at[inv_v.at[j]], zsem)
        for j in range(_ZROUNDS)
    ]

    gbufs = (gbuf0, gbuf1)
    gsems = (gsem0, gsem1)
    ssems = (ssem0, ssem1)
    gh = [None] * _VROUNDS
    sh = [None] * _VROUNDS
    gh[0] = pltpu.async_copy(roi_hbm.at[vsrc_v.at[0]], gbufs[0], gsems[0])
    for j in range(_VROUNDS):
        cur = j & 1
        nxt = cur ^ 1
        gh[j].wait()
        sh[j] = pltpu.async_copy(gbufs[cur], out_hbm.at[vdst_v.at[j]],
                                 ssems[cur])
        if j + 1 < _VROUNDS:
            if j >= 1:
                sh[j - 1].wait()
            gh[j + 1] = pltpu.async_copy(roi_hbm.at[vsrc_v.at[j + 1]],
                                         gbufs[nxt], gsems[nxt])
    sh[_VROUNDS - 1].wait()
    for h in zcopies:
        h.wait()


def _sc_gather(roi_chunks, zrow, inv3, vsrc3, vdst3):
    mesh = plsc.VectorSubcoreMesh(core_axis_name="c", subcore_axis_name="s")
    run = pl.kernel(
        _sc_gather_body,
        out_type=jax.ShapeDtypeStruct((_ND * _HW, _MEM_DIM), jnp.float32),
        mesh=mesh,
        scratch_types=[
            pltpu.VMEM((_ZROUNDS, _CB), jnp.int32),
            pltpu.VMEM((_VROUNDS, _CB), jnp.int32),
            pltpu.VMEM((_VROUNDS, _CB), jnp.int32),
            pltpu.VMEM((_CB, _MEM_DIM), jnp.float32),
            pltpu.VMEM((_CB, _MEM_DIM), jnp.float32),
            pltpu.VMEM((_CB, _MEM_DIM), jnp.float32),
            pltpu.SemaphoreType.DMA,
            pltpu.SemaphoreType.DMA,
            pltpu.SemaphoreType.DMA,
            pltpu.SemaphoreType.DMA,
            pltpu.SemaphoreType.DMA,
        ],
        interpret=_INTERPRET,
    )
    return run(roi_chunks, zrow, inv3, vsrc3, vdst3)


def kernel(prop_class, prop_feature, prop_deltas, prop_scale,
           roi_class, roi_feature, roi_deltas, roi_scale):
    f32 = jnp.float32
    ptab = jnp.concatenate(
        [prop_deltas, prop_scale[:, None],
         jnp.zeros((_N, 3), f32)], axis=1)
    rtab = jnp.concatenate(
        [roi_deltas, roi_scale[:, None],
         jnp.zeros((_N, 3), f32)], axis=1)

    (pf, ps, rs, zrow, inv, vsrc, vdst) = _prep(
        prop_class, roi_class, prop_feature, ptab, rtab)

    roi_chunks = roi_feature.transpose(2, 3, 0, 1).reshape(_N * _HW, _MEM_DIM)
    roi_mem2 = _sc_gather(roi_chunks, zrow,
                          inv.reshape(_NW, _ZROUNDS, _CB),
                          vsrc.reshape(_NW, _VROUNDS, _CB),
                          vdst.reshape(_NW, _VROUNDS, _CB))
    roi_mem = (roi_mem2
               .reshape(_NUM_CLASSES, _ROI_SIZE, _ROI_SIZE,
                        _NUM_INSTANCE, _MEM_DIM)
               .transpose(0, 3, 4, 1, 2))

    return (
        pf.reshape(_NUM_CLASSES, _NUM_INSTANCE, _MEM_DIM),
        ps[:, :4].reshape(_NUM_CLASSES, _NUM_INSTANCE, 4),
        ps[:, 4].reshape(_NUM_CLASSES, _NUM_INSTANCE),
        roi_mem,
        rs[:, :4].reshape(_NUM_CLASSES, _NUM_INSTANCE, 4),
        rs[:, 4].reshape(_NUM_CLASSES, _NUM_INSTANCE),
    )


# speedup vs baseline: 13.0206x; 1.0417x over previous
import functools

import jax
import jax.numpy as jnp
from jax import lax
from jax.experimental import pallas as pl
from jax.experimental.pallas import tpu as pltpu
from jax.experimental.pallas import tpu_sc as plsc

_NUM_CLASSES = 80
_NUM_INSTANCE = 32
_MEM_DIM = 256
_ROI_SIZE = 7
_N = 512
_ND = _NUM_CLASSES * _NUM_INSTANCE
_HW = _ROI_SIZE * _ROI_SIZE

_NC = 2
_NS = 16
_NW = _NC * _NS
_CB = 112
_ZROUNDS = (_ND * _HW) // (_NW * _CB)
_VROUNDS = (_N * _HW) // (_NW * _CB)

_INTERPRET = False


def _side(cls_r, cls_c):
    i32 = jnp.int32
    ii = lax.broadcasted_iota(i32, (_N, _N), 0)
    jj = lax.broadcasted_iota(i32, (_N, _N), 1)
    d_r = lax.broadcasted_iota(i32, (_ND, 1), 0)
    jD = lax.broadcasted_iota(i32, (_ND, _N), 1)
    eq = cls_r == cls_c
    rank_r = jnp.sum((eq & (jj < ii)).astype(i32), axis=1, keepdims=True)
    rank_c = jnp.sum((eq & (ii < jj)).astype(i32), axis=0, keepdims=True)
    dest_r = (cls_r - 1) * _NUM_INSTANCE + (rank_r & (_NUM_INSTANCE - 1))
    dest_c = (cls_c - 1) * _NUM_INSTANCE + (rank_c & (_NUM_INSTANCE - 1))
    samedest = dest_r == dest_c
    wsrc_r = jnp.max(jnp.where(samedest, jj, -1), axis=1, keepdims=True)
    dmat = d_r == dest_c
    srctab_r = jnp.max(jnp.where(dmat, jD, -1), axis=1, keepdims=True)
    return dest_r, wsrc_r, srctab_r, jD


def _prep_idx_body(rcls_r_ref, rcls_c_ref, zrow_ref,
                   inv_ref, vsrc_ref, vdst_ref):
    i32 = jnp.int32
    dest_r, wsrc_r, srctab_r, _ = _side(rcls_r_ref[...], rcls_c_ref[...])
    zrow_ref[...] = jnp.zeros_like(zrow_ref)

    d_r = lax.broadcasted_iota(i32, (_ND, 1), 0)
    valid = srctab_r >= 0
    j0 = jnp.min(jnp.where(valid, _ND, d_r), axis=0, keepdims=True)
    hw_d = lax.broadcasted_iota(i32, (_ND, _HW), 1)
    hw_n = lax.broadcasted_iota(i32, (_N, _HW), 1)

    def chunk(d, hw):
        return (d >> 5) * (_HW * _NUM_INSTANCE) + hw * _NUM_INSTANCE + (d & 31)

    dd = jnp.where(valid, j0, d_r)
    inv_ref[...] = chunk(dd, hw_d)
    vsrc_ref[...] = hw_n * _N + wsrc_r
    vdst_ref[...] = chunk(dest_r, hw_n)


def _prep_idx(rcls):
    i32 = jnp.int32
    f32 = jnp.float32
    out_shapes = (
        jax.ShapeDtypeStruct((_CB, _MEM_DIM), f32),
        jax.ShapeDtypeStruct((_ND, _HW), i32),
        jax.ShapeDtypeStruct((_N, _HW), i32),
        jax.ShapeDtypeStruct((_N, _HW), i32),
    )
    return pl.pallas_call(
        _prep_idx_body,
        out_shape=out_shapes,
        interpret=_INTERPRET,
    )(rcls.reshape(_N, 1), rcls.reshape(1, _N))


def _prep_dense_body(pcls_r_ref, pcls_c_ref, rcls_r_ref, rcls_c_ref,
                     pfeat_ref, ptab_ref, rtab_ref,
                     pf_ref, ps_ref, rs_ref):
    _, _, srctab_p, jD = _side(pcls_r_ref[...], pcls_c_ref[...])
    _, _, srctab_r, _ = _side(rcls_r_ref[...], rcls_c_ref[...])
    onehot_p = (srctab_p == jD).astype(jnp.float32)
    onehot_r = (srctab_r == jD).astype(jnp.float32)

    pf_ref[...] = jnp.dot(onehot_p, pfeat_ref[...],
                          preferred_element_type=jnp.float32)
    ps_ref[...] = jnp.dot(onehot_p, ptab_ref[...],
                          preferred_element_type=jnp.float32)
    rs_ref[...] = jnp.dot(onehot_r, rtab_ref[...],
                          preferred_element_type=jnp.float32)


def _prep_dense(pcls, rcls, pfeat, ptab, rtab):
    f32 = jnp.float32
    out_shapes = (
        jax.ShapeDtypeStruct((_ND, _MEM_DIM), f32),
        jax.ShapeDtypeStruct((_ND, 8), f32),
        jax.ShapeDtypeStruct((_ND, 8), f32),
    )
    return pl.pallas_call(
        _prep_dense_body,
        out_shape=out_shapes,
        interpret=_INTERPRET,
    )(pcls.reshape(_N, 1), pcls.reshape(1, _N),
      rcls.reshape(_N, 1), rcls.reshape(1, _N), pfeat, ptab, rtab)


def _sc_gather_body(roi_hbm, zrow_hbm, inv_hbm, vsrc_hbm, vdst_hbm, out_hbm,
                    inv_v, vsrc_v, vdst_v, zbuf, gbuf0, gbuf1,
                    zsem, gsem0, gsem1, ssem0, ssem1):
    wid = lax.axis_index("s") * _NC + lax.axis_index("c")
    pltpu.sync_copy(inv_hbm.at[wid], inv_v)
    pltpu.sync_copy(vsrc_hbm.at[wid], vsrc_v)
    pltpu.sync_copy(vdst_hbm.at[wid], vdst_v)
    pltpu.sync_copy(zrow_hbm, zbuf)

    zcopies = [
        pltpu.async_copy(zbuf, out_hbm.at[inv_v.at[j]], zsem)
        for j in range(_ZROUNDS)
    ]

    gbufs = (gbuf0, gbuf1)
    gsems = (gsem0, gsem1)
    ssems = (ssem0, ssem1)
    gh = [None] * _VROUNDS
    sh = [None] * _VROUNDS
    gh[0] = pltpu.async_copy(roi_hbm.at[vsrc_v.at[0]], gbufs[0], gsems[0])
    for j in range(_VROUNDS):
        cur = j & 1
        nxt = cur ^ 1
        gh[j].wait()
        sh[j] = pltpu.async_copy(gbufs[cur], out_hbm.at[vdst_v.at[j]],
                                 ssems[cur])
        if j + 1 < _VROUNDS:
            if j >= 1:
                sh[j - 1].wait()
            gh[j + 1] = pltpu.async_copy(roi_hbm.at[vsrc_v.at[j + 1]],
                                         gbufs[nxt], gsems[nxt])
    sh[_VROUNDS - 1].wait()
    for h in zcopies:
        h.wait()


def _sc_gather(roi_chunks, zrow, inv3, vsrc3, vdst3):
    mesh = plsc.VectorSubcoreMesh(core_axis_name="c", subcore_axis_name="s")
    run = pl.kernel(
        _sc_gather_body,
        out_type=jax.ShapeDtypeStruct((_ND * _HW, _MEM_DIM), jnp.float32),
        mesh=mesh,
        scratch_types=[
            pltpu.VMEM((_ZROUNDS, _CB), jnp.int32),
            pltpu.VMEM((_VROUNDS, _CB), jnp.int32),
            pltpu.VMEM((_VROUNDS, _CB), jnp.int32),
            pltpu.VMEM((_CB, _MEM_DIM), jnp.float32),
            pltpu.VMEM((_CB, _MEM_DIM), jnp.float32),
            pltpu.VMEM((_CB, _MEM_DIM), jnp.float32),
            pltpu.SemaphoreType.DMA,
            pltpu.SemaphoreType.DMA,
            pltpu.SemaphoreType.DMA,
            pltpu.SemaphoreType.DMA,
            pltpu.SemaphoreType.DMA,
        ],
        interpret=_INTERPRET,
    )
    return run(roi_chunks, zrow, inv3, vsrc3, vdst3)


def kernel(prop_class, prop_feature, prop_deltas, prop_scale,
           roi_class, roi_feature, roi_deltas, roi_scale):
    f32 = jnp.float32
    ptab = jnp.concatenate(
        [prop_deltas, prop_scale[:, None],
         jnp.zeros((_N, 3), f32)], axis=1)
    rtab = jnp.concatenate(
        [roi_deltas, roi_scale[:, None],
         jnp.zeros((_N, 3), f32)], axis=1)

    (zrow, inv, vsrc, vdst) = _prep_idx(roi_class)
    (pf, ps, rs) = _prep_dense(prop_class, roi_class, prop_feature, ptab, rtab)

    roi_chunks = roi_feature.transpose(2, 3, 0, 1).reshape(_N * _HW, _MEM_DIM)
    roi_mem2 = _sc_gather(roi_chunks, zrow,
                          inv.reshape(_NW, _ZROUNDS, _CB),
                          vsrc.reshape(_NW, _VROUNDS, _CB),
                          vdst.reshape(_NW, _VROUNDS, _CB))
    roi_mem = (roi_mem2
               .reshape(_NUM_CLASSES, _ROI_SIZE, _ROI_SIZE,
                        _NUM_INSTANCE, _MEM_DIM)
               .transpose(0, 3, 4, 1, 2))

    return (
        pf.reshape(_NUM_CLASSES, _NUM_INSTANCE, _MEM_DIM),
        ps[:, :4].reshape(_NUM_CLASSES, _NUM_INSTANCE, 4),
        ps[:, 4].reshape(_NUM_CLASSES, _NUM_INSTANCE),
        roi_mem,
        rs[:, :4].reshape(_NUM_CLASSES, _NUM_INSTANCE, 4),
        rs[:, 4].reshape(_NUM_CLASSES, _NUM_INSTANCE),
    )


# speedup vs baseline: 17.0296x; 1.3079x over previous
import functools

import jax
import jax.numpy as jnp
from jax import lax
from jax.experimental import pallas as pl
from jax.experimental.pallas import tpu as pltpu
from jax.experimental.pallas import tpu_sc as plsc

_NUM_CLASSES = 80
_NUM_INSTANCE = 32
_MEM_DIM = 256
_ROI_SIZE = 7
_N = 512
_ND = _NUM_CLASSES * _NUM_INSTANCE
_HW = _ROI_SIZE * _ROI_SIZE

_NC = 2
_NS = 16
_NW = _NC * _NS
_CB = 112
_ZROUNDS = (_ND * _HW) // (_NW * _CB)
_VROUNDS = (_N * _HW) // (_NW * _CB)

_INTERPRET = False


def _side(cls_r, cls_c):
    i32 = jnp.int32
    ii = lax.broadcasted_iota(i32, (_N, _N), 0)
    jj = lax.broadcasted_iota(i32, (_N, _N), 1)
    d_r = lax.broadcasted_iota(i32, (_ND, 1), 0)
    jD = lax.broadcasted_iota(i32, (_ND, _N), 1)
    eq = cls_r == cls_c
    rank_r = jnp.sum((eq & (jj < ii)).astype(i32), axis=1, keepdims=True)
    rank_c = jnp.sum((eq & (ii < jj)).astype(i32), axis=0, keepdims=True)
    dest_r = (cls_r - 1) * _NUM_INSTANCE + (rank_r & (_NUM_INSTANCE - 1))
    dest_c = (cls_c - 1) * _NUM_INSTANCE + (rank_c & (_NUM_INSTANCE - 1))
    samedest = dest_r == dest_c
    wsrc_r = jnp.max(jnp.where(samedest, jj, -1), axis=1, keepdims=True)
    dmat = d_r == dest_c
    srctab_r = jnp.max(jnp.where(dmat, jD, -1), axis=1, keepdims=True)
    return dest_r, wsrc_r, srctab_r, jD


def _prep_idx_body(rcls_r_ref, rcls_c_ref, zrow_ref,
                   inv_ref, cnt_ref, vsrc_ref, vdst_ref):
    i32 = jnp.int32
    dest_r, wsrc_r, srctab_r, _ = _side(rcls_r_ref[...], rcls_c_ref[...])
    zrow_ref[...] = jnp.zeros_like(zrow_ref)

    d_r = lax.broadcasted_iota(i32, (_ND, 1), 0)
    valid = srctab_r >= 0
    j0 = jnp.min(jnp.where(valid, _ND, d_r), axis=0, keepdims=True)
    hw_n = lax.broadcasted_iota(i32, (_N, _HW), 1)

    def chunk(d, hw):
        return (d >> 5) * (_HW * _NUM_INSTANCE) + hw * _NUM_INSTANCE + (d & 31)

    GL0 = _ND // _NW
    invg = (~valid).astype(i32).reshape(_NW, GL0)
    lt1 = lax.broadcasted_iota(i32, (GL0, GL0), 0)
    lt2 = lax.broadcasted_iota(i32, (GL0, GL0), 1)
    ltri = (lt1 <= lt2).astype(jnp.float32)
    cs = jnp.dot(invg.astype(jnp.float32), ltri,
                 preferred_element_type=jnp.float32).astype(i32)
    pos = (cs - invg)[:, None, :]
    GL = _ND // _NW
    kk = lax.broadcasted_iota(i32, (_NW, GL, GL), 1)
    dl = lax.broadcasted_iota(i32, (_NW, GL, GL), 2)
    gg = lax.broadcasted_iota(i32, (_NW, GL, GL), 0)
    m = (pos == kk) & (invg[:, None, :] != 0)
    crow = jnp.sum(jnp.where(m, gg * GL + dl, 0), axis=2)
    ninv = cs[:, -1:]
    kk2 = lax.broadcasted_iota(i32, (_NW, GL), 1)
    crow = jnp.where(kk2 < ninv, crow, j0)
    hw_g = lax.broadcasted_iota(i32, (_NW, GL, _HW), 2)
    inv_ref[...] = chunk(crow[:, :, None], hw_g)
    rounds = (ninv * _HW + (_CB - 1)) // _CB
    cnt_ref[...] = jnp.broadcast_to(rounds, (_NW, 16))

    vsrc_ref[...] = hw_n * _N + wsrc_r
    vdst_ref[...] = chunk(dest_r, hw_n)


def _prep_idx(rcls):
    i32 = jnp.int32
    f32 = jnp.float32
    out_shapes = (
        jax.ShapeDtypeStruct((_CB, _MEM_DIM), f32),
        jax.ShapeDtypeStruct((_NW, _ND // _NW, _HW), i32),
        jax.ShapeDtypeStruct((_NW, 16), i32),
        jax.ShapeDtypeStruct((_N, _HW), i32),
        jax.ShapeDtypeStruct((_N, _HW), i32),
    )
    return pl.pallas_call(
        _prep_idx_body,
        out_shape=out_shapes,
        interpret=_INTERPRET,
    )(rcls.reshape(_N, 1), rcls.reshape(1, _N))


def _prep_dense_body(pcls_r_ref, pcls_c_ref, rcls_r_ref, rcls_c_ref,
                     pfeat_ref, ptab_ref, rtab_ref,
                     pf_ref, ps_ref, rs_ref):
    _, _, srctab_p, jD = _side(pcls_r_ref[...], pcls_c_ref[...])
    _, _, srctab_r, _ = _side(rcls_r_ref[...], rcls_c_ref[...])
    onehot_p = (srctab_p == jD).astype(jnp.float32)
    onehot_r = (srctab_r == jD).astype(jnp.float32)

    pf_ref[...] = jnp.dot(onehot_p, pfeat_ref[...],
                          preferred_element_type=jnp.float32)
    ps_ref[...] = jnp.dot(onehot_p, ptab_ref[...],
                          preferred_element_type=jnp.float32)
    rs_ref[...] = jnp.dot(onehot_r, rtab_ref[...],
                          preferred_element_type=jnp.float32)


def _prep_dense(pcls, rcls, pfeat, ptab, rtab):
    f32 = jnp.float32
    out_shapes = (
        jax.ShapeDtypeStruct((_ND, _MEM_DIM), f32),
        jax.ShapeDtypeStruct((_ND, 8), f32),
        jax.ShapeDtypeStruct((_ND, 8), f32),
    )
    return pl.pallas_call(
        _prep_dense_body,
        out_shape=out_shapes,
        interpret=_INTERPRET,
    )(pcls.reshape(_N, 1), pcls.reshape(1, _N),
      rcls.reshape(_N, 1), rcls.reshape(1, _N), pfeat, ptab, rtab)


def _sc_gather_body(roi_hbm, zrow_hbm, inv_hbm, cnt_hbm, vsrc_hbm, vdst_hbm,
                    out_hbm,
                    inv_v, cnt_v, vsrc_v, vdst_v, zbuf, gbuf0, gbuf1,
                    zsem, gsem0, gsem1, ssem0, ssem1):
    wid = lax.axis_index("s") * _NC + lax.axis_index("c")
    pltpu.sync_copy(inv_hbm.at[wid], inv_v)
    pltpu.sync_copy(cnt_hbm.at[wid], cnt_v)
    pltpu.sync_copy(vsrc_hbm.at[wid], vsrc_v)
    pltpu.sync_copy(vdst_hbm.at[wid], vdst_v)
    pltpu.sync_copy(zrow_hbm, zbuf)
    nz = cnt_v[...][0]

    for j in range(_ZROUNDS):
        @pl.when(j < nz)
        def _():
            pltpu.async_copy(zbuf, out_hbm.at[inv_v.at[j]], zsem)

    gbufs = (gbuf0, gbuf1)
    gsems = (gsem0, gsem1)
    ssems = (ssem0, ssem1)
    gh = [None] * _VROUNDS
    sh = [None] * _VROUNDS
    gh[0] = pltpu.async_copy(roi_hbm.at[vsrc_v.at[0]], gbufs[0], gsems[0])
    for j in range(_VROUNDS):
        cur = j & 1
        nxt = cur ^ 1
        gh[j].wait()
        sh[j] = pltpu.async_copy(gbufs[cur], out_hbm.at[vdst_v.at[j]],
                                 ssems[cur])
        if j + 1 < _VROUNDS:
            if j >= 1:
                sh[j - 1].wait()
            gh[j + 1] = pltpu.async_copy(roi_hbm.at[vsrc_v.at[j + 1]],
                                         gbufs[nxt], gsems[nxt])
    sh[_VROUNDS - 1].wait()

    for j in range(_ZROUNDS):
        @pl.when(j < nz)
        def _():
            pltpu.make_async_copy(zbuf, out_hbm.at[inv_v.at[0]], zsem).wait()


def _sc_gather(roi_chunks, zrow, inv3, cnts, vsrc3, vdst3):
    mesh = plsc.VectorSubcoreMesh(core_axis_name="c", subcore_axis_name="s")
    run = pl.kernel(
        _sc_gather_body,
        out_type=jax.ShapeDtypeStruct((_ND * _HW, _MEM_DIM), jnp.float32),
        mesh=mesh,
        scratch_types=[
            pltpu.VMEM((_ZROUNDS, _CB), jnp.int32),
            pltpu.VMEM((16,), jnp.int32),
            pltpu.VMEM((_VROUNDS, _CB), jnp.int32),
            pltpu.VMEM((_VROUNDS, _CB), jnp.int32),
            pltpu.VMEM((_CB, _MEM_DIM), jnp.float32),
            pltpu.VMEM((_CB, _MEM_DIM), jnp.float32),
            pltpu.VMEM((_CB, _MEM_DIM), jnp.float32),
            pltpu.SemaphoreType.DMA,
            pltpu.SemaphoreType.DMA,
            pltpu.SemaphoreType.DMA,
            pltpu.SemaphoreType.DMA,
            pltpu.SemaphoreType.DMA,
        ],
        interpret=_INTERPRET,
    )
    return run(roi_chunks, zrow, inv3, cnts, vsrc3, vdst3)


def kernel(prop_class, prop_feature, prop_deltas, prop_scale,
           roi_class, roi_feature, roi_deltas, roi_scale):
    f32 = jnp.float32
    ptab = jnp.concatenate(
        [prop_deltas, prop_scale[:, None],
         jnp.zeros((_N, 3), f32)], axis=1)
    rtab = jnp.concatenate(
        [roi_deltas, roi_scale[:, None],
         jnp.zeros((_N, 3), f32)], axis=1)

    (zrow, inv, cnts, vsrc, vdst) = _prep_idx(roi_class)
    (pf, ps, rs) = _prep_dense(prop_class, roi_class, prop_feature, ptab, rtab)

    roi_chunks = roi_feature.transpose(2, 3, 0, 1).reshape(_N * _HW, _MEM_DIM)
    roi_mem2 = _sc_gather(roi_chunks, zrow,
                          inv.reshape(_NW, _ZROUNDS, _CB), cnts,
                          vsrc.reshape(_NW, _VROUNDS, _CB),
                          vdst.reshape(_NW, _VROUNDS, _CB))
    roi_mem = (roi_mem2
               .reshape(_NUM_CLASSES, _ROI_SIZE, _ROI_SIZE,
                        _NUM_INSTANCE, _MEM_DIM)
               .transpose(0, 3, 4, 1, 2))

    return (
        pf.reshape(_NUM_CLASSES, _NUM_INSTANCE, _MEM_DIM),
        ps[:, :4].reshape(_NUM_CLASSES, _NUM_INSTANCE, 4),
        ps[:, 4].reshape(_NUM_CLASSES, _NUM_INSTANCE),
        roi_mem,
        rs[:, :4].reshape(_NUM_CLASSES, _NUM_INSTANCE, 4),
        rs[:, 4].reshape(_NUM_CLASSES, _NUM_INSTANCE),
    )


# speedup vs baseline: 17.5668x; 1.0315x over previous
import functools

import jax
import jax.numpy as jnp
from jax import lax
from jax.experimental import pallas as pl
from jax.experimental.pallas import tpu as pltpu
from jax.experimental.pallas import tpu_sc as plsc

_NUM_CLASSES = 80
_NUM_INSTANCE = 32
_MEM_DIM = 256
_ROI_SIZE = 7
_N = 512
_ND = _NUM_CLASSES * _NUM_INSTANCE
_HW = _ROI_SIZE * _ROI_SIZE

_NC = 2
_NS = 16
_NW = _NC * _NS
_CB = 112
_ZROUNDS = (_ND * _HW) // (_NW * _CB)
_VROUNDS = (_N * _HW) // (_NW * _CB)

_INTERPRET = False


def _rank_side(cls_r, cls_c):
    i32 = jnp.int32
    ii = lax.broadcasted_iota(i32, (_N, _N), 0)
    jj = lax.broadcasted_iota(i32, (_N, _N), 1)
    eq = cls_r == cls_c
    rank_r = jnp.sum((eq & (jj < ii)).astype(i32), axis=1, keepdims=True)
    rank_c = jnp.sum((eq & (ii < jj)).astype(i32), axis=0, keepdims=True)
    dest_r = (cls_r - 1) * _NUM_INSTANCE + (rank_r & (_NUM_INSTANCE - 1))
    dest_c = (cls_c - 1) * _NUM_INSTANCE + (rank_c & (_NUM_INSTANCE - 1))
    samedest = dest_r == dest_c
    wsrc_r = jnp.max(jnp.where(samedest, jj, -1), axis=1, keepdims=True)
    return dest_r, dest_c, wsrc_r


def _side(cls_r, cls_c):
    i32 = jnp.int32
    d_r = lax.broadcasted_iota(i32, (_ND, 1), 0)
    jD = lax.broadcasted_iota(i32, (_ND, _N), 1)
    dest_r, dest_c, wsrc_r = _rank_side(cls_r, cls_c)
    dmat = d_r == dest_c
    srctab_r = jnp.max(jnp.where(dmat, jD, -1), axis=1, keepdims=True)
    return dest_r, wsrc_r, srctab_r, jD


def _prep_idx_body(rcls_r_ref, rcls_c_ref, zrow_ref,
                   inv_ref, cnt_ref, vsrc_ref, vdst_ref):
    i32 = jnp.int32
    dest_r, _, wsrc_r = _rank_side(rcls_r_ref[...], rcls_c_ref[...])
    zrow_ref[...] = jnp.zeros_like(zrow_ref)

    d_r = lax.broadcasted_iota(i32, (_ND, 1), 0)
    ciota = lax.broadcasted_iota(i32, (_N, _NUM_CLASSES), 1) + 1
    ccnt = jnp.sum((ciota == rcls_r_ref[...]).astype(i32),
                   axis=0, keepdims=True)
    cmatch = (d_r >> 5) == lax.broadcasted_iota(i32, (_ND, _NUM_CLASSES), 1)
    cnt_d = jnp.sum(cmatch.astype(i32) * ccnt, axis=1, keepdims=True)
    valid = (d_r & 31) < cnt_d
    j0 = jnp.min(jnp.where(valid, _ND, d_r), axis=0, keepdims=True)
    hw_n = lax.broadcasted_iota(i32, (_N, _HW), 1)

    def chunk(d, hw):
        return (d >> 5) * (_HW * _NUM_INSTANCE) + hw * _NUM_INSTANCE + (d & 31)

    GL0 = _ND // _NW
    invg = (~valid).astype(i32).reshape(_NW, GL0)
    lt1 = lax.broadcasted_iota(i32, (GL0, GL0), 0)
    lt2 = lax.broadcasted_iota(i32, (GL0, GL0), 1)
    ltri = (lt1 <= lt2).astype(jnp.float32)
    cs = jnp.dot(invg.astype(jnp.float32), ltri,
                 preferred_element_type=jnp.float32).astype(i32)
    pos = (cs - invg)[:, None, :]
    GL = _ND // _NW
    kk = lax.broadcasted_iota(i32, (_NW, GL, GL), 1)
    dl = lax.broadcasted_iota(i32, (_NW, GL, GL), 2)
    gg = lax.broadcasted_iota(i32, (_NW, GL, GL), 0)
    m = (pos == kk) & (invg[:, None, :] != 0)
    crow = jnp.sum(jnp.where(m, gg * GL + dl, 0), axis=2)
    ninv = cs[:, -1:]
    kk2 = lax.broadcasted_iota(i32, (_NW, GL), 1)
    crow = jnp.where(kk2 < ninv, crow, j0)
    hw_g = lax.broadcasted_iota(i32, (_NW, GL, _HW), 2)
    inv_ref[...] = chunk(crow[:, :, None], hw_g)
    rounds = (ninv * _HW + (_CB - 1)) // _CB
    cnt_ref[...] = jnp.broadcast_to(rounds, (_NW, 16))

    vsrc_ref[...] = hw_n * _N + wsrc_r
    vdst_ref[...] = chunk(dest_r, hw_n)


def _prep_idx(rcls):
    i32 = jnp.int32
    f32 = jnp.float32
    out_shapes = (
        jax.ShapeDtypeStruct((_CB, _MEM_DIM), f32),
        jax.ShapeDtypeStruct((_NW, _ND // _NW, _HW), i32),
        jax.ShapeDtypeStruct((_NW, 16), i32),
        jax.ShapeDtypeStruct((_N, _HW), i32),
        jax.ShapeDtypeStruct((_N, _HW), i32),
    )
    return pl.pallas_call(
        _prep_idx_body,
        out_shape=out_shapes,
        interpret=_INTERPRET,
    )(rcls.reshape(_N, 1), rcls.reshape(1, _N))


def _prep_dense_body(pcls_r_ref, pcls_c_ref, rcls_r_ref, rcls_c_ref,
                     pfeat_ref, ptab_ref, rtab_ref,
                     pf_ref, ps_ref, rs_ref):
    _, _, srctab_p, jD = _side(pcls_r_ref[...], pcls_c_ref[...])
    _, _, srctab_r, _ = _side(rcls_r_ref[...], rcls_c_ref[...])
    onehot_p = (srctab_p == jD).astype(jnp.float32)
    onehot_r = (srctab_r == jD).astype(jnp.float32)

    pf_ref[...] = jnp.dot(onehot_p, pfeat_ref[...],
                          preferred_element_type=jnp.float32)
    ps_ref[...] = jnp.dot(onehot_p, ptab_ref[...],
                          preferred_element_type=jnp.float32)
    rs_ref[...] = jnp.dot(onehot_r, rtab_ref[...],
                          preferred_element_type=jnp.float32)


def _prep_dense(pcls, rcls, pfeat, ptab, rtab):
    f32 = jnp.float32
    out_shapes = (
        jax.ShapeDtypeStruct((_ND, _MEM_DIM), f32),
        jax.ShapeDtypeStruct((_ND, 8), f32),
        jax.ShapeDtypeStruct((_ND, 8), f32),
    )
    return pl.pallas_call(
        _prep_dense_body,
        out_shape=out_shapes,
        interpret=_INTERPRET,
    )(pcls.reshape(_N, 1), pcls.reshape(1, _N),
      rcls.reshape(_N, 1), rcls.reshape(1, _N), pfeat, ptab, rtab)


def _sc_gather_body(roi_hbm, zrow_hbm, inv_hbm, cnt_hbm, vsrc_hbm, vdst_hbm,
                    out_hbm,
                    inv_v, cnt_v, vsrc_v, vdst_v, zbuf, gbuf0, gbuf1,
                    zsem, gsem0, gsem1, ssem0, ssem1):
    wid = lax.axis_index("s") * _NC + lax.axis_index("c")
    st = [pltpu.async_copy(inv_hbm.at[wid], inv_v, gsem0),
          pltpu.async_copy(cnt_hbm.at[wid], cnt_v, gsem1),
          pltpu.async_copy(vsrc_hbm.at[wid], vsrc_v, ssem0),
          pltpu.async_copy(vdst_hbm.at[wid], vdst_v, ssem1),
          pltpu.async_copy(zrow_hbm, zbuf, zsem)]
    for h in st:
        h.wait()
    nz = cnt_v[...][0]

    for j in range(_ZROUNDS):
        @pl.when(j < nz)
        def _():
            pltpu.async_copy(zbuf, out_hbm.at[inv_v.at[j]], zsem)

    gbufs = (gbuf0, gbuf1)
    gsems = (gsem0, gsem1)
    ssems = (ssem0, ssem1)
    gh = [None] * _VROUNDS
    sh = [None] * _VROUNDS
    gh[0] = pltpu.async_copy(roi_hbm.at[vsrc_v.at[0]], gbufs[0], gsems[0])
    for j in range(_VROUNDS):
        cur = j & 1
        nxt = cur ^ 1
        gh[j].wait()
        sh[j] = pltpu.async_copy(gbufs[cur], out_hbm.at[vdst_v.at[j]],
                                 ssems[cur])
        if j + 1 < _VROUNDS:
            if j >= 1:
                sh[j - 1].wait()
            gh[j + 1] = pltpu.async_copy(roi_hbm.at[vsrc_v.at[j + 1]],
                                         gbufs[nxt], gsems[nxt])
    sh[_VROUNDS - 1].wait()

    for j in range(_ZROUNDS):
        @pl.when(j < nz)
        def _():
            pltpu.make_async_copy(zbuf, out_hbm.at[inv_v.at[0]], zsem).wait()


def _sc_gather(roi_chunks, zrow, inv3, cnts, vsrc3, vdst3):
    mesh = plsc.VectorSubcoreMesh(core_axis_name="c", subcore_axis_name="s")
    run = pl.kernel(
        _sc_gather_body,
        out_type=jax.ShapeDtypeStruct((_ND * _HW, _MEM_DIM), jnp.float32),
        mesh=mesh,
        scratch_types=[
            pltpu.VMEM((_ZROUNDS, _CB), jnp.int32),
            pltpu.VMEM((16,), jnp.int32),
            pltpu.VMEM((_VROUNDS, _CB), jnp.int32),
            pltpu.VMEM((_VROUNDS, _CB), jnp.int32),
            pltpu.VMEM((_CB, _MEM_DIM), jnp.float32),
            pltpu.VMEM((_CB, _MEM_DIM), jnp.float32),
            pltpu.VMEM((_CB, _MEM_DIM), jnp.float32),
            pltpu.SemaphoreType.DMA,
            pltpu.SemaphoreType.DMA,
            pltpu.SemaphoreType.DMA,
            pltpu.SemaphoreType.DMA,
            pltpu.SemaphoreType.DMA,
        ],
        interpret=_INTERPRET,
    )
    return run(roi_chunks, zrow, inv3, cnts, vsrc3, vdst3)


def kernel(prop_class, prop_feature, prop_deltas, prop_scale,
           roi_class, roi_feature, roi_deltas, roi_scale):
    f32 = jnp.float32
    ptab = jnp.concatenate(
        [prop_deltas, prop_scale[:, None],
         jnp.zeros((_N, 3), f32)], axis=1)
    rtab = jnp.concatenate(
        [roi_deltas, roi_scale[:, None],
         jnp.zeros((_N, 3), f32)], axis=1)

    (zrow, inv, cnts, vsrc, vdst) = _prep_idx(roi_class)
    (pf, ps, rs) = _prep_dense(prop_class, roi_class, prop_feature, ptab, rtab)

    roi_chunks = roi_feature.transpose(2, 3, 0, 1).reshape(_N * _HW, _MEM_DIM)
    roi_mem2 = _sc_gather(roi_chunks, zrow,
                          inv.reshape(_NW, _ZROUNDS, _CB), cnts,
                          vsrc.reshape(_NW, _VROUNDS, _CB),
                          vdst.reshape(_NW, _VROUNDS, _CB))
    roi_mem = (roi_mem2
               .reshape(_NUM_CLASSES, _ROI_SIZE, _ROI_SIZE,
                        _NUM_INSTANCE, _MEM_DIM)
               .transpose(0, 3, 4, 1, 2))

    return (
        pf.reshape(_NUM_CLASSES, _NUM_INSTANCE, _MEM_DIM),
        ps[:, :4].reshape(_NUM_CLASSES, _NUM_INSTANCE, 4),
        ps[:, 4].reshape(_NUM_CLASSES, _NUM_INSTANCE),
        roi_mem,
        rs[:, :4].reshape(_NUM_CLASSES, _NUM_INSTANCE, 4),
        rs[:, 4].reshape(_NUM_CLASSES, _NUM_INSTANCE),
    )


# speedup vs baseline: 17.9012x; 1.0190x over previous
import functools

import jax
import jax.numpy as jnp
from jax import lax
from jax.experimental import pallas as pl
from jax.experimental.pallas import tpu as pltpu
from jax.experimental.pallas import tpu_sc as plsc

_NUM_CLASSES = 80
_NUM_INSTANCE = 32
_MEM_DIM = 256
_ROI_SIZE = 7
_N = 512
_ND = _NUM_CLASSES * _NUM_INSTANCE
_HW = _ROI_SIZE * _ROI_SIZE

_NC = 2
_NS = 16
_NW = _NC * _NS
_CB = 112
_ZROUNDS = (_ND * _HW) // (_NW * _CB)
_VROUNDS = (_N * _HW) // (_NW * _CB)

_INTERPRET = False


def _rank_side(cls_r, cls_c):
    i32 = jnp.int32
    ii = lax.broadcasted_iota(i32, (_N, _N), 0)
    jj = lax.broadcasted_iota(i32, (_N, _N), 1)
    eq = cls_r == cls_c
    rank_r = jnp.sum((eq & (jj < ii)).astype(i32), axis=1, keepdims=True)
    rank_c = jnp.sum((eq & (ii < jj)).astype(i32), axis=0, keepdims=True)
    dest_r = (cls_r - 1) * _NUM_INSTANCE + (rank_r & (_NUM_INSTANCE - 1))
    dest_c = (cls_c - 1) * _NUM_INSTANCE + (rank_c & (_NUM_INSTANCE - 1))
    samedest = dest_r == dest_c
    wsrc_r = jnp.max(jnp.where(samedest, jj, -1), axis=1, keepdims=True)
    return dest_r, dest_c, wsrc_r


def _side(cls_r, cls_c):
    i32 = jnp.int32
    d_r = lax.broadcasted_iota(i32, (_ND, 1), 0)
    jD = lax.broadcasted_iota(i32, (_ND, _N), 1)
    dest_r, dest_c, wsrc_r = _rank_side(cls_r, cls_c)
    dmat = d_r == dest_c
    srctab_r = jnp.max(jnp.where(dmat, jD, -1), axis=1, keepdims=True)
    return dest_r, wsrc_r, srctab_r, jD


def _prep_idx_body(rcls_r_ref, rcls_c_ref, zrow_ref,
                   inv_ref, cnt_ref, vsrc_ref, vdst_ref):
    i32 = jnp.int32
    dest_r, _, wsrc_r = _rank_side(rcls_r_ref[...], rcls_c_ref[...])
    zrow_ref[...] = jnp.zeros_like(zrow_ref)

    d_r = lax.broadcasted_iota(i32, (_ND, 1), 0)
    ciota = lax.broadcasted_iota(i32, (_N, _NUM_CLASSES), 1) + 1
    ccnt = jnp.sum((ciota == rcls_r_ref[...]).astype(i32),
                   axis=0, keepdims=True)
    cmatch = (d_r >> 5) == lax.broadcasted_iota(i32, (_ND, _NUM_CLASSES), 1)
    cnt_d = jnp.sum(cmatch.astype(i32) * ccnt, axis=1, keepdims=True)
    valid = (d_r & 31) < cnt_d
    j0 = jnp.min(jnp.where(valid, _ND, d_r), axis=0, keepdims=True)
    hw_n = lax.broadcasted_iota(i32, (_N, _HW), 1)

    def chunk(d, hw):
        return (d >> 5) * (_HW * _NUM_INSTANCE) + hw * _NUM_INSTANCE + (d & 31)

    GL0 = _ND // _NW
    invg = (~valid).astype(i32).reshape(_NW, GL0)
    lt1 = lax.broadcasted_iota(i32, (GL0, GL0), 0)
    lt2 = lax.broadcasted_iota(i32, (GL0, GL0), 1)
    ltri = (lt1 <= lt2).astype(jnp.float32)
    cs = jnp.dot(invg.astype(jnp.float32), ltri,
                 preferred_element_type=jnp.float32).astype(i32)
    pos = (cs - invg)[:, None, :]
    GL = _ND // _NW
    kk = lax.broadcasted_iota(i32, (_NW, GL, GL), 1)
    dl = lax.broadcasted_iota(i32, (_NW, GL, GL), 2)
    gg = lax.broadcasted_iota(i32, (_NW, GL, GL), 0)
    m = (pos == kk) & (invg[:, None, :] != 0)
    crow = jnp.sum(jnp.where(m, gg * GL + dl, 0), axis=2)
    ninv = cs[:, -1:]
    kk2 = lax.broadcasted_iota(i32, (_NW, GL), 1)
    crow = jnp.where(kk2 < ninv, crow, j0)
    rounds = (ninv * _HW + (_CB - 1)) // _CB
    cnt_ref[...] = jnp.broadcast_to(rounds, (_NW, 16))

    f32 = jnp.float32

    def round_layout(vals, nrows, nrounds):
        rr = lax.broadcasted_iota(i32, (_NW, nrounds, _CB), 1)
        bb = lax.broadcasted_iota(i32, (_NW, nrounds, _CB), 2)
        f = rr * _CB + bb
        k = f // _HW
        hwf = f - k * _HW
        k0_3d = (rr * _CB) // _HW
        dk = k - k0_3d
        kio = lax.broadcasted_iota(i32, (nrows, nrounds), 0)
        rio = lax.broadcasted_iota(i32, (nrows, nrounds), 1)
        k0 = (rio * _CB) // _HW
        vals_f = vals.astype(f32)
        sel = [jnp.dot(vals_f, (kio == k0 + delta).astype(f32),
                       preferred_element_type=f32)[:, :, None]
               for delta in range(4)]
        v = jnp.where(dk == 0, sel[0],
                      jnp.where(dk == 1, sel[1],
                                jnp.where(dk == 2, sel[2], sel[3])))
        return v.astype(i32), hwf

    crow_sel, hw_z = round_layout(crow, GL, _ZROUNDS)
    inv_ref[...] = chunk(crow_sel, hw_z)

    NI = _N // _NW
    wsrc_g = wsrc_r.reshape(_NW, NI)
    dest_g = dest_r.reshape(_NW, NI)
    wsrc_sel, hw_v = round_layout(wsrc_g, NI, _VROUNDS)
    dest_sel, _ = round_layout(dest_g, NI, _VROUNDS)
    vsrc_ref[...] = hw_v * _N + wsrc_sel
    vdst_ref[...] = chunk(dest_sel, hw_v)


def _prep_idx(rcls):
    i32 = jnp.int32
    f32 = jnp.float32
    out_shapes = (
        jax.ShapeDtypeStruct((_CB, _MEM_DIM), f32),
        jax.ShapeDtypeStruct((_NW, _ZROUNDS, _CB), i32),
        jax.ShapeDtypeStruct((_NW, 16), i32),
        jax.ShapeDtypeStruct((_NW, _VROUNDS, _CB), i32),
        jax.ShapeDtypeStruct((_NW, _VROUNDS, _CB), i32),
    )
    return pl.pallas_call(
        _prep_idx_body,
        out_shape=out_shapes,
        interpret=_INTERPRET,
    )(rcls.reshape(_N, 1), rcls.reshape(1, _N))


def _prep_dense_body(pcls_r_ref, pcls_c_ref, rcls_r_ref, rcls_c_ref,
                     pfeat_ref, ptab_ref, rtab_ref,
                     pf_ref, ps_ref, rs_ref):
    _, _, srctab_p, jD = _side(pcls_r_ref[...], pcls_c_ref[...])
    _, _, srctab_r, _ = _side(rcls_r_ref[...], rcls_c_ref[...])
    onehot_p = (srctab_p == jD).astype(jnp.float32)
    onehot_r = (srctab_r == jD).astype(jnp.float32)

    pf_ref[...] = jnp.dot(onehot_p, pfeat_ref[...],
                          preferred_element_type=jnp.float32)
    ps_ref[...] = jnp.dot(onehot_p, ptab_ref[...],
                          preferred_element_type=jnp.float32)
    rs_ref[...] = jnp.dot(onehot_r, rtab_ref[...],
                          preferred_element_type=jnp.float32)


def _prep_dense(pcls, rcls, pfeat, ptab, rtab):
    f32 = jnp.float32
    out_shapes = (
        jax.ShapeDtypeStruct((_ND, _MEM_DIM), f32),
        jax.ShapeDtypeStruct((_ND, 8), f32),
        jax.ShapeDtypeStruct((_ND, 8), f32),
    )
    return pl.pallas_call(
        _prep_dense_body,
        out_shape=out_shapes,
        interpret=_INTERPRET,
    )(pcls.reshape(_N, 1), pcls.reshape(1, _N),
      rcls.reshape(_N, 1), rcls.reshape(1, _N), pfeat, ptab, rtab)


def _sc_gather_body(roi_hbm, zrow_hbm, inv_hbm, cnt_hbm, vsrc_hbm, vdst_hbm,
                    out_hbm,
                    inv_v, cnt_v, vsrc_v, vdst_v, zbuf, gbuf0, gbuf1,
                    zsem, gsem0, gsem1, ssem0, ssem1):
    wid = lax.axis_index("s") * _NC + lax.axis_index("c")
    st = [pltpu.async_copy(inv_hbm.at[wid], inv_v, gsem0),
          pltpu.async_copy(cnt_hbm.at[wid], cnt_v, gsem1),
          pltpu.async_copy(vsrc_hbm.at[wid], vsrc_v, ssem0),
          pltpu.async_copy(vdst_hbm.at[wid], vdst_v, ssem1),
          pltpu.async_copy(zrow_hbm, zbuf, zsem)]
    for h in st:
        h.wait()
    nz = cnt_v[...][0]

    for j in range(_ZROUNDS):
        @pl.when(j < nz)
        def _():
            pltpu.async_copy(zbuf, out_hbm.at[inv_v.at[j]], zsem)

    gbufs = (gbuf0, gbuf1)
    gsems = (gsem0, gsem1)
    ssems = (ssem0, ssem1)
    gh = [None] * _VROUNDS
    sh = [None] * _VROUNDS
    gh[0] = pltpu.async_copy(roi_hbm.at[vsrc_v.at[0]], gbufs[0], gsems[0])
    for j in range(_VROUNDS):
        cur = j & 1
        nxt = cur ^ 1
        gh[j].wait()
        sh[j] = pltpu.async_copy(gbufs[cur], out_hbm.at[vdst_v.at[j]],
                                 ssems[cur])
        if j + 1 < _VROUNDS:
            if j >= 1:
                sh[j - 1].wait()
            gh[j + 1] = pltpu.async_copy(roi_hbm.at[vsrc_v.at[j + 1]],
                                         gbufs[nxt], gsems[nxt])
    sh[_VROUNDS - 1].wait()

    for j in range(_ZROUNDS):
        @pl.when(j < nz)
        def _():
            pltpu.make_async_copy(zbuf, out_hbm.at[inv_v.at[0]], zsem).wait()


def _sc_gather(roi_chunks, zrow, inv3, cnts, vsrc3, vdst3):
    mesh = plsc.VectorSubcoreMesh(core_axis_name="c", subcore_axis_name="s")
    run = pl.kernel(
        _sc_gather_body,
        out_type=jax.ShapeDtypeStruct((_ND * _HW, _MEM_DIM), jnp.float32),
        mesh=mesh,
        scratch_types=[
            pltpu.VMEM((_ZROUNDS, _CB), jnp.int32),
            pltpu.VMEM((16,), jnp.int32),
            pltpu.VMEM((_VROUNDS, _CB), jnp.int32),
            pltpu.VMEM((_VROUNDS, _CB), jnp.int32),
            pltpu.VMEM((_CB, _MEM_DIM), jnp.float32),
            pltpu.VMEM((_CB, _MEM_DIM), jnp.float32),
            pltpu.VMEM((_CB, _MEM_DIM), jnp.float32),
            pltpu.SemaphoreType.DMA,
            pltpu.SemaphoreType.DMA,
            pltpu.SemaphoreType.DMA,
            pltpu.SemaphoreType.DMA,
            pltpu.SemaphoreType.DMA,
        ],
        interpret=_INTERPRET,
    )
    return run(roi_chunks, zrow, inv3, cnts, vsrc3, vdst3)


def kernel(prop_class, prop_feature, prop_deltas, prop_scale,
           roi_class, roi_feature, roi_deltas, roi_scale):
    f32 = jnp.float32
    ptab = jnp.concatenate(
        [prop_deltas, prop_scale[:, None],
         jnp.zeros((_N, 3), f32)], axis=1)
    rtab = jnp.concatenate(
        [roi_deltas, roi_scale[:, None],
         jnp.zeros((_N, 3), f32)], axis=1)

    (zrow, inv, cnts, vsrc, vdst) = _prep_idx(roi_class)
    (pf, ps, rs) = _prep_dense(prop_class, roi_class, prop_feature, ptab, rtab)

    roi_chunks = roi_feature.transpose(2, 3, 0, 1).reshape(_N * _HW, _MEM_DIM)
    roi_mem2 = _sc_gather(roi_chunks, zrow, inv, cnts, vsrc, vdst)
    roi_mem = (roi_mem2
               .reshape(_NUM_CLASSES, _ROI_SIZE, _ROI_SIZE,
                        _NUM_INSTANCE, _MEM_DIM)
               .transpose(0, 3, 4, 1, 2))

    return (
        pf.reshape(_NUM_CLASSES, _NUM_INSTANCE, _MEM_DIM),
        ps[:, :4].reshape(_NUM_CLASSES, _NUM_INSTANCE, 4),
        ps[:, 4].reshape(_NUM_CLASSES, _NUM_INSTANCE),
        roi_mem,
        rs[:, :4].reshape(_NUM_CLASSES, _NUM_INSTANCE, 4),
        rs[:, 4].reshape(_NUM_CLASSES, _NUM_INSTANCE),
    )


# speedup vs baseline: 18.2817x; 1.0213x over previous
import functools

import jax
import jax.numpy as jnp
from jax import lax
from jax.experimental import pallas as pl
from jax.experimental.pallas import tpu as pltpu
from jax.experimental.pallas import tpu_sc as plsc

_NUM_CLASSES = 80
_NUM_INSTANCE = 32
_MEM_DIM = 256
_ROI_SIZE = 7
_N = 512
_ND = _NUM_CLASSES * _NUM_INSTANCE
_HW = _ROI_SIZE * _ROI_SIZE

_NC = 2
_NS = 16
_NW = _NC * _NS
_CB = 112
_ZROUNDS = (_ND * _HW) // (_NW * _CB)
_VROUNDS = (_N * _HW) // (_NW * _CB)

_INTERPRET = False


def _rank_side(cls_r, cls_c):
    i32 = jnp.int32
    ii = lax.broadcasted_iota(i32, (_N, _N), 0)
    jj = lax.broadcasted_iota(i32, (_N, _N), 1)
    eq = cls_r == cls_c
    rank_r = jnp.sum((eq & (jj < ii)).astype(i32), axis=1, keepdims=True)
    rank_c = jnp.sum((eq & (ii < jj)).astype(i32), axis=0, keepdims=True)
    dest_r = (cls_r - 1) * _NUM_INSTANCE + (rank_r & (_NUM_INSTANCE - 1))
    dest_c = (cls_c - 1) * _NUM_INSTANCE + (rank_c & (_NUM_INSTANCE - 1))
    samedest = dest_r == dest_c
    wsrc_r = jnp.max(jnp.where(samedest, jj, -1), axis=1, keepdims=True)
    return dest_r, dest_c, wsrc_r


def _side(cls_r, cls_c):
    i32 = jnp.int32
    d_r = lax.broadcasted_iota(i32, (_ND, 1), 0)
    jD = lax.broadcasted_iota(i32, (_ND, _N), 1)
    dest_r, dest_c, wsrc_r = _rank_side(cls_r, cls_c)
    dmat = d_r == dest_c
    srctab_r = jnp.max(jnp.where(dmat, jD, -1), axis=1, keepdims=True)
    return dest_r, wsrc_r, srctab_r, jD


def _prep_idx_body(rcls_r_ref, rcls_c_ref, zrow_ref,
                   inv_ref, cnt_ref, vsrc_ref, vdst_ref):
    i32 = jnp.int32
    dest_r, _, wsrc_r = _rank_side(rcls_r_ref[...], rcls_c_ref[...])
    zrow_ref[...] = jnp.zeros_like(zrow_ref)

    d_r = lax.broadcasted_iota(i32, (_ND, 1), 0)
    ciota = lax.broadcasted_iota(i32, (_N, _NUM_CLASSES), 1) + 1
    ccnt = jnp.sum((ciota == rcls_r_ref[...]).astype(i32),
                   axis=0, keepdims=True)
    cmatch = (d_r >> 5) == lax.broadcasted_iota(i32, (_ND, _NUM_CLASSES), 1)
    cnt_d = jnp.sum(cmatch.astype(i32) * ccnt, axis=1, keepdims=True)
    valid = (d_r & 31) < cnt_d
    j0 = jnp.min(jnp.where(valid, _ND, d_r), axis=0, keepdims=True)
    hw_n = lax.broadcasted_iota(i32, (_N, _HW), 1)

    def chunk(d, hw):
        return (d >> 5) * (_HW * _NUM_INSTANCE) + hw * _NUM_INSTANCE + (d & 31)

    GL0 = _ND // _NW
    invg = (~valid).astype(i32).reshape(_NW, GL0)
    lt1 = lax.broadcasted_iota(i32, (GL0, GL0), 0)
    lt2 = lax.broadcasted_iota(i32, (GL0, GL0), 1)
    ltri = (lt1 <= lt2).astype(jnp.float32)
    cs = jnp.dot(invg.astype(jnp.float32), ltri,
                 precision=lax.Precision.HIGHEST,
                 preferred_element_type=jnp.float32).astype(i32)
    pos = (cs - invg)[:, None, :]
    GL = _ND // _NW
    kk = lax.broadcasted_iota(i32, (_NW, GL, GL), 1)
    dl = lax.broadcasted_iota(i32, (_NW, GL, GL), 2)
    gg = lax.broadcasted_iota(i32, (_NW, GL, GL), 0)
    m = (pos == kk) & (invg[:, None, :] != 0)
    crow = jnp.sum(jnp.where(m, gg * GL + dl, 0), axis=2)
    ninv = cs[:, -1:]
    kk2 = lax.broadcasted_iota(i32, (_NW, GL), 1)
    crow = jnp.where(kk2 < ninv, crow, j0)
    rounds = (ninv * _HW + (_CB - 1)) // _CB
    cnt_ref[...] = jnp.broadcast_to(rounds, (_NW, 16))

    f32 = jnp.float32

    def round_layout(vals, nrows, nrounds):
        rr = lax.broadcasted_iota(i32, (_NW, nrounds, _CB), 1)
        bb = lax.broadcasted_iota(i32, (_NW, nrounds, _CB), 2)
        f = rr * _CB + bb
        k = f // _HW
        hwf = f - k * _HW
        k0_3d = (rr * _CB) // _HW
        dk = k - k0_3d
        kio = lax.broadcasted_iota(i32, (nrows, nrounds), 0)
        rio = lax.broadcasted_iota(i32, (nrows, nrounds), 1)
        k0 = (rio * _CB) // _HW
        vals_f = vals.astype(f32)
        sel = [jnp.dot(vals_f, (kio == k0 + delta).astype(f32),
                       precision=lax.Precision.HIGHEST,
                       preferred_element_type=f32)[:, :, None]
               for delta in range(4)]
        v = jnp.where(dk == 0, sel[0],
                      jnp.where(dk == 1, sel[1],
                                jnp.where(dk == 2, sel[2], sel[3])))
        return v.astype(i32), hwf

    crow_sel, hw_z = round_layout(crow, GL, _ZROUNDS)
    inv_ref[...] = chunk(crow_sel, hw_z)

    NI = _N // _NW
    wsrc_g = wsrc_r.reshape(_NW, NI)
    dest_g = dest_r.reshape(_NW, NI)
    wsrc_sel, hw_v = round_layout(wsrc_g, NI, _VROUNDS)
    dest_sel, _ = round_layout(dest_g, NI, _VROUNDS)
    vsrc_ref[...] = hw_v * _N + wsrc_sel
    vdst_ref[...] = chunk(dest_sel, hw_v)


def _prep_idx(rcls):
    i32 = jnp.int32
    f32 = jnp.float32
    out_shapes = (
        jax.ShapeDtypeStruct((_CB, _MEM_DIM), f32),
        jax.ShapeDtypeStruct((_NW, _ZROUNDS, _CB), i32),
        jax.ShapeDtypeStruct((_NW, 16), i32),
        jax.ShapeDtypeStruct((_NW, _VROUNDS, _CB), i32),
        jax.ShapeDtypeStruct((_NW, _VROUNDS, _CB), i32),
    )
    return pl.pallas_call(
        _prep_idx_body,
        out_shape=out_shapes,
        interpret=_INTERPRET,
    )(rcls.reshape(_N, 1), rcls.reshape(1, _N))


def _prep_dense_body(pcls_r_ref, pcls_c_ref, rcls_r_ref, rcls_c_ref,
                     pfeat_ref, ptab_ref, rtab_ref,
                     pf_ref, ps_ref, rs_ref):
    _, _, srctab_p, jD = _side(pcls_r_ref[...], pcls_c_ref[...])
    _, _, srctab_r, _ = _side(rcls_r_ref[...], rcls_c_ref[...])
    onehot_p = (srctab_p == jD).astype(jnp.float32)
    onehot_r = (srctab_r == jD).astype(jnp.float32)

    pf_ref[...] = jnp.dot(onehot_p, pfeat_ref[...],
                          precision=lax.Precision.HIGHEST,
                          preferred_element_type=jnp.float32)
    ps_ref[...] = jnp.dot(onehot_p, ptab_ref[...],
                          precision=lax.Precision.HIGHEST,
                          preferred_element_type=jnp.float32)
    rs_ref[...] = jnp.dot(onehot_r, rtab_ref[...],
                          precision=lax.Precision.HIGHEST,
                          preferred_element_type=jnp.float32)


def _prep_dense(pcls, rcls, pfeat, ptab, rtab):
    f32 = jnp.float32
    out_shapes = (
        jax.ShapeDtypeStruct((_ND, _MEM_DIM), f32),
        jax.ShapeDtypeStruct((_ND, 8), f32),
        jax.ShapeDtypeStruct((_ND, 8), f32),
    )
    return pl.pallas_call(
        _prep_dense_body,
        out_shape=out_shapes,
        interpret=_INTERPRET,
    )(pcls.reshape(_N, 1), pcls.reshape(1, _N),
      rcls.reshape(_N, 1), rcls.reshape(1, _N), pfeat, ptab, rtab)


def _sc_gather_body(roi_hbm, zrow_hbm, inv_hbm, cnt_hbm, vsrc_hbm, vdst_hbm,
                    out_hbm,
                    inv_v, cnt_v, vsrc_v, vdst_v, zbuf, gbuf0, gbuf1,
                    zsem, gsem0, gsem1, ssem0, ssem1):
    wid = lax.axis_index("s") * _NC + lax.axis_index("c")
    st = [pltpu.async_copy(inv_hbm.at[wid], inv_v, gsem0),
          pltpu.async_copy(cnt_hbm.at[wid], cnt_v, gsem1),
          pltpu.async_copy(vsrc_hbm.at[wid], vsrc_v, ssem0),
          pltpu.async_copy(vdst_hbm.at[wid], vdst_v, ssem1),
          pltpu.async_copy(zrow_hbm, zbuf, zsem)]
    for h in st:
        h.wait()
    nz = cnt_v[...][0]

    for j in range(_ZROUNDS):
        @pl.when(j < nz)
        def _():
            pltpu.async_copy(zbuf, out_hbm.at[inv_v.at[j]], zsem)

    gbufs = (gbuf0, gbuf1)
    gsems = (gsem0, gsem1)
    ssems = (ssem0, ssem1)
    gh = [None] * _VROUNDS
    sh = [None] * _VROUNDS
    gh[0] = pltpu.async_copy(roi_hbm.at[vsrc_v.at[0]], gbufs[0], gsems[0])
    for j in range(_VROUNDS):
        cur = j & 1
        nxt = cur ^ 1
        gh[j].wait()
        sh[j] = pltpu.async_copy(gbufs[cur], out_hbm.at[vdst_v.at[j]],
                                 ssems[cur])
        if j + 1 < _VROUNDS:
            if j >= 1:
                sh[j - 1].wait()
            gh[j + 1] = pltpu.async_copy(roi_hbm.at[vsrc_v.at[j + 1]],
                                         gbufs[nxt], gsems[nxt])
    sh[_VROUNDS - 1].wait()

    for j in range(_ZROUNDS):
        @pl.when(j < nz)
        def _():
            pltpu.make_async_copy(zbuf, out_hbm.at[inv_v.at[0]], zsem).wait()


def _sc_gather(roi_chunks, zrow, inv3, cnts, vsrc3, vdst3):
    mesh = plsc.VectorSubcoreMesh(core_axis_name="c", subcore_axis_name="s")
    run = pl.kernel(
        _sc_gather_body,
        out_type=jax.ShapeDtypeStruct((_ND * _HW, _MEM_DIM), jnp.float32),
        mesh=mesh,
        scratch_types=[
            pltpu.VMEM((_ZROUNDS, _CB), jnp.int32),
            pltpu.VMEM((16,), jnp.int32),
            pltpu.VMEM((_VROUNDS, _CB), jnp.int32),
            pltpu.VMEM((_VROUNDS, _CB), jnp.int32),
            pltpu.VMEM((_CB, _MEM_DIM), jnp.float32),
            pltpu.VMEM((_CB, _MEM_DIM), jnp.float32),
            pltpu.VMEM((_CB, _MEM_DIM), jnp.float32),
            pltpu.SemaphoreType.DMA,
            pltpu.SemaphoreType.DMA,
            pltpu.SemaphoreType.DMA,
            pltpu.SemaphoreType.DMA,
            pltpu.SemaphoreType.DMA,
        ],
        interpret=_INTERPRET,
    )
    return run(roi_chunks, zrow, inv3, cnts, vsrc3, vdst3)


def kernel(prop_class, prop_feature, prop_deltas, prop_scale,
           roi_class, roi_feature, roi_deltas, roi_scale):
    f32 = jnp.float32
    ptab = jnp.concatenate(
        [prop_deltas, prop_scale[:, None],
         jnp.zeros((_N, 3), f32)], axis=1)
    rtab = jnp.concatenate(
        [roi_deltas, roi_scale[:, None],
         jnp.zeros((_N, 3), f32)], axis=1)

    (zrow, inv, cnts, vsrc, vdst) = _prep_idx(roi_class)
    (pf, ps, rs) = _prep_dense(prop_class, roi_class, prop_feature, ptab, rtab)

    roi_chunks = roi_feature.transpose(2, 3, 0, 1).reshape(_N * _HW, _MEM_DIM)
    roi_mem2 = _sc_gather(roi_chunks, zrow, inv, cnts, vsrc, vdst)
    roi_mem = (roi_mem2
               .reshape(_NUM_CLASSES, _ROI_SIZE, _ROI_SIZE,
                        _NUM_INSTANCE, _MEM_DIM)
               .transpose(0, 3, 4, 1, 2))

    return (
        pf.reshape(_NUM_CLASSES, _NUM_INSTANCE, _MEM_DIM),
        ps[:, :4].reshape(_NUM_CLASSES, _NUM_INSTANCE, 4),
        ps[:, 4].reshape(_NUM_CLASSES, _NUM_INSTANCE),
        roi_mem,
        rs[:, :4].reshape(_NUM_CLASSES, _NUM_INSTANCE, 4),
        rs[:, 4].reshape(_NUM_CLASSES, _NUM_INSTANCE),
    )


# speedup vs baseline: 19.3239x; 1.0570x over previous
import functools

import jax
import jax.numpy as jnp
from jax import lax
from jax.experimental import pallas as pl
from jax.experimental.pallas import tpu as pltpu
from jax.experimental.pallas import tpu_sc as plsc

_NUM_CLASSES = 80
_NUM_INSTANCE = 32
_MEM_DIM = 256
_ROI_SIZE = 7
_N = 512
_ND = _NUM_CLASSES * _NUM_INSTANCE
_HW = _ROI_SIZE * _ROI_SIZE

_NC = 2
_NS = 16
_NW = _NC * _NS
_CB = 112
_ZROUNDS = (_ND * _HW) // (_NW * _CB)
_VROUNDS = (_N * _HW) // (_NW * _CB)

_INTERPRET = False


def _rank_side(cls_r, cls_c):
    i32 = jnp.int32
    ii = lax.broadcasted_iota(i32, (_N, _N), 0)
    jj = lax.broadcasted_iota(i32, (_N, _N), 1)
    eq = cls_r == cls_c
    rank_r = jnp.sum((eq & (jj < ii)).astype(i32), axis=1, keepdims=True)
    rank_c = jnp.sum((eq & (ii < jj)).astype(i32), axis=0, keepdims=True)
    dest_r = (cls_r - 1) * _NUM_INSTANCE + (rank_r & (_NUM_INSTANCE - 1))
    dest_c = (cls_c - 1) * _NUM_INSTANCE + (rank_c & (_NUM_INSTANCE - 1))
    samedest = dest_r == dest_c
    wsrc_r = jnp.max(jnp.where(samedest, jj, -1), axis=1, keepdims=True)
    return dest_r, dest_c, wsrc_r


def _side(cls_r, cls_c):
    i32 = jnp.int32
    d_r = lax.broadcasted_iota(i32, (_ND, 1), 0)
    jD = lax.broadcasted_iota(i32, (_ND, _N), 1)
    dest_r, dest_c, wsrc_r = _rank_side(cls_r, cls_c)
    dmat = d_r == dest_c
    srctab_r = jnp.max(jnp.where(dmat, jD, -1), axis=1, keepdims=True)
    return dest_r, wsrc_r, srctab_r, jD


def _prep_idx_body(rcls_r_ref, rcls_c_ref, zrow_ref,
                   inv_ref, cnt_ref, vsrc_ref, vdst_ref):
    i32 = jnp.int32
    dest_r, _, wsrc_r = _rank_side(rcls_r_ref[...], rcls_c_ref[...])
    zrow_ref[...] = jnp.zeros_like(zrow_ref)

    d_r = lax.broadcasted_iota(i32, (_ND, 1), 0)
    ciota = lax.broadcasted_iota(i32, (_N, _NUM_CLASSES), 1) + 1
    ccnt = jnp.sum((ciota == rcls_r_ref[...]).astype(i32),
                   axis=0, keepdims=True)
    cmatch = (d_r >> 5) == lax.broadcasted_iota(i32, (_ND, _NUM_CLASSES), 1)
    cnt_d = jnp.sum(cmatch.astype(i32) * ccnt, axis=1, keepdims=True)
    valid = (d_r & 31) < cnt_d
    j0 = jnp.min(jnp.where(valid, _ND, d_r), axis=0, keepdims=True)
    hw_n = lax.broadcasted_iota(i32, (_N, _HW), 1)

    def chunk(d, hw):
        return (d >> 5) * (_HW * _NUM_INSTANCE) + hw * _NUM_INSTANCE + (d & 31)

    GL0 = _ND // _NW
    invg = (~valid).astype(i32).reshape(_NW, GL0)
    lt1 = lax.broadcasted_iota(i32, (GL0, GL0), 0)
    lt2 = lax.broadcasted_iota(i32, (GL0, GL0), 1)
    ltri = (lt1 <= lt2).astype(jnp.float32)
    cs = jnp.dot(invg.astype(jnp.float32), ltri,
                 precision=lax.Precision.HIGHEST,
                 preferred_element_type=jnp.float32).astype(i32)
    pos = (cs - invg)[:, None, :]
    GL = _ND // _NW
    kk = lax.broadcasted_iota(i32, (_NW, GL, GL), 1)
    dl = lax.broadcasted_iota(i32, (_NW, GL, GL), 2)
    gg = lax.broadcasted_iota(i32, (_NW, GL, GL), 0)
    m = (pos == kk) & (invg[:, None, :] != 0)
    crow = jnp.sum(jnp.where(m, gg * GL + dl, 0), axis=2)
    ninv = cs[:, -1:]
    kk2 = lax.broadcasted_iota(i32, (_NW, GL), 1)
    crow = jnp.where(kk2 < ninv, crow, j0)
    rounds = (ninv * _HW + (_CB - 1)) // _CB
    cnt_ref[...] = jnp.broadcast_to(rounds, (_NW, 16))

    f32 = jnp.float32

    def round_layout(vals, nrows, nrounds):
        rr = lax.broadcasted_iota(i32, (_NW, nrounds, _CB), 1)
        bb = lax.broadcasted_iota(i32, (_NW, nrounds, _CB), 2)
        f = rr * _CB + bb
        k = f // _HW
        hwf = f - k * _HW
        k0_3d = (rr * _CB) // _HW
        dk = k - k0_3d
        kio = lax.broadcasted_iota(i32, (nrows, nrounds), 0)
        rio = lax.broadcasted_iota(i32, (nrows, nrounds), 1)
        k0 = (rio * _CB) // _HW
        vals_f = vals.astype(f32)
        sel = [jnp.dot(vals_f, (kio == k0 + delta).astype(f32),
                       precision=lax.Precision.HIGHEST,
                       preferred_element_type=f32)[:, :, None]
               for delta in range(4)]
        v = jnp.where(dk == 0, sel[0],
                      jnp.where(dk == 1, sel[1],
                                jnp.where(dk == 2, sel[2], sel[3])))
        return v.astype(i32), hwf

    crow_sel, hw_z = round_layout(crow, GL, _ZROUNDS)
    inv_ref[...] = chunk(crow_sel, hw_z)

    NI = _N // _NW
    wsrc_g = wsrc_r.reshape(_NW, NI)
    dest_g = dest_r.reshape(_NW, NI)
    wsrc_sel, hw_v = round_layout(wsrc_g, NI, _VROUNDS)
    dest_sel, _ = round_layout(dest_g, NI, _VROUNDS)
    vsrc_ref[...] = hw_v * _N + wsrc_sel
    vdst_ref[...] = chunk(dest_sel, hw_v)


def _prep_idx(rcls):
    i32 = jnp.int32
    f32 = jnp.float32
    out_shapes = (
        jax.ShapeDtypeStruct((_CB, _MEM_DIM), f32),
        jax.ShapeDtypeStruct((_NW, _ZROUNDS, _CB), i32),
        jax.ShapeDtypeStruct((_NW, 16), i32),
        jax.ShapeDtypeStruct((_NW, _VROUNDS, _CB), i32),
        jax.ShapeDtypeStruct((_NW, _VROUNDS, _CB), i32),
    )
    return pl.pallas_call(
        _prep_idx_body,
        out_shape=out_shapes,
        interpret=_INTERPRET,
    )(rcls.reshape(_N, 1), rcls.reshape(1, _N))


def _prep_dense_body(pcls_r_ref, pcls_c_ref, rcls_r_ref, rcls_c_ref,
                     pfeat_ref, ptab_ref, rtab_ref,
                     pf_ref, ps_ref, rs_ref):
    _, _, srctab_p, jD = _side(pcls_r_ref[...], pcls_c_ref[...])
    _, _, srctab_r, _ = _side(rcls_r_ref[...], rcls_c_ref[...])
    onehot_p = (srctab_p == jD).astype(jnp.float32)
    onehot_r = (srctab_r == jD).astype(jnp.float32)

    pf_ref[...] = jnp.dot(onehot_p, pfeat_ref[...],
                          precision=lax.Precision.HIGHEST,
                          preferred_element_type=jnp.float32)
    ps_ref[...] = jnp.dot(onehot_p, ptab_ref[...],
                          precision=lax.Precision.HIGHEST,
                          preferred_element_type=jnp.float32)
    rs_ref[...] = jnp.dot(onehot_r, rtab_ref[...],
                          precision=lax.Precision.HIGHEST,
                          preferred_element_type=jnp.float32)


def _prep_dense(pcls, rcls, pfeat, ptab, rtab):
    f32 = jnp.float32
    out_shapes = (
        jax.ShapeDtypeStruct((_ND, _MEM_DIM), f32),
        jax.ShapeDtypeStruct((_ND, 8), f32),
        jax.ShapeDtypeStruct((_ND, 8), f32),
    )
    return pl.pallas_call(
        _prep_dense_body,
        out_shape=out_shapes,
        interpret=_INTERPRET,
    )(pcls.reshape(_N, 1), pcls.reshape(1, _N),
      rcls.reshape(_N, 1), rcls.reshape(1, _N), pfeat, ptab, rtab)


def _sc_gather_body(roi_hbm, zrow_hbm, inv_hbm, cnt_hbm, vsrc_hbm, vdst_hbm,
                    out_hbm,
                    inv_v, cnt_v, vsrc_v, vdst_v, zbuf, gbuf0, gbuf1,
                    zsem, gsem0, gsem1, ssem0, ssem1):
    wid = lax.axis_index("s") * _NC + lax.axis_index("c")
    st = [pltpu.async_copy(inv_hbm.at[wid], inv_v, gsem0),
          pltpu.async_copy(cnt_hbm.at[wid], cnt_v, gsem1),
          pltpu.async_copy(vsrc_hbm.at[wid], vsrc_v, ssem0),
          pltpu.async_copy(vdst_hbm.at[wid], vdst_v, ssem1),
          pltpu.async_copy(zrow_hbm, zbuf, zsem)]
    for h in st:
        h.wait()
    nz = cnt_v[...][0]

    def _zfire(j, carry):
        pltpu.async_copy(zbuf, out_hbm.at[inv_v.at[j]], zsem)
        return carry

    lax.fori_loop(0, nz, _zfire, 0)

    gbufs = (gbuf0, gbuf1)
    gsems = (gsem0, gsem1)
    ssems = (ssem0, ssem1)
    gh = [None] * _VROUNDS
    sh = [None] * _VROUNDS
    gh[0] = pltpu.async_copy(roi_hbm.at[vsrc_v.at[0]], gbufs[0], gsems[0])
    for j in range(_VROUNDS):
        cur = j & 1
        nxt = cur ^ 1
        gh[j].wait()
        sh[j] = pltpu.async_copy(gbufs[cur], out_hbm.at[vdst_v.at[j]],
                                 ssems[cur])
        if j + 1 < _VROUNDS:
            if j >= 1:
                sh[j - 1].wait()
            gh[j + 1] = pltpu.async_copy(roi_hbm.at[vsrc_v.at[j + 1]],
                                         gbufs[nxt], gsems[nxt])
    sh[_VROUNDS - 1].wait()

    def _zdrain(j, carry):
        pltpu.make_async_copy(zbuf, out_hbm.at[inv_v.at[0]], zsem).wait()
        return carry

    lax.fori_loop(0, nz, _zdrain, 0)


def _sc_gather(roi_chunks, zrow, inv3, cnts, vsrc3, vdst3):
    mesh = plsc.VectorSubcoreMesh(core_axis_name="c", subcore_axis_name="s")
    run = pl.kernel(
        _sc_gather_body,
        out_type=jax.ShapeDtypeStruct((_ND * _HW, _MEM_DIM), jnp.float32),
        mesh=mesh,
        scratch_types=[
            pltpu.VMEM((_ZROUNDS, _CB), jnp.int32),
            pltpu.VMEM((16,), jnp.int32),
            pltpu.VMEM((_VROUNDS, _CB), jnp.int32),
            pltpu.VMEM((_VROUNDS, _CB), jnp.int32),
            pltpu.VMEM((_CB, _MEM_DIM), jnp.float32),
            pltpu.VMEM((_CB, _MEM_DIM), jnp.float32),
            pltpu.VMEM((_CB, _MEM_DIM), jnp.float32),
            pltpu.SemaphoreType.DMA,
            pltpu.SemaphoreType.DMA,
            pltpu.SemaphoreType.DMA,
            pltpu.SemaphoreType.DMA,
            pltpu.SemaphoreType.DMA,
        ],
        interpret=_INTERPRET,
    )
    return run(roi_chunks, zrow, inv3, cnts, vsrc3, vdst3)


def kernel(prop_class, prop_feature, prop_deltas, prop_scale,
           roi_class, roi_feature, roi_deltas, roi_scale):
    f32 = jnp.float32
    ptab = jnp.concatenate(
        [prop_deltas, prop_scale[:, None],
         jnp.zeros((_N, 3), f32)], axis=1)
    rtab = jnp.concatenate(
        [roi_deltas, roi_scale[:, None],
         jnp.zeros((_N, 3), f32)], axis=1)

    (zrow, inv, cnts, vsrc, vdst) = _prep_idx(roi_class)
    (pf, ps, rs) = _prep_dense(prop_class, roi_class, prop_feature, ptab, rtab)

    roi_chunks = roi_feature.transpose(2, 3, 0, 1).reshape(_N * _HW, _MEM_DIM)
    roi_mem2 = _sc_gather(roi_chunks, zrow, inv, cnts, vsrc, vdst)
    roi_mem = (roi_mem2
               .reshape(_NUM_CLASSES, _ROI_SIZE, _ROI_SIZE,
                        _NUM_INSTANCE, _MEM_DIM)
               .transpose(0, 3, 4, 1, 2))

    return (
        pf.reshape(_NUM_CLASSES, _NUM_INSTANCE, _MEM_DIM),
        ps[:, :4].reshape(_NUM_CLASSES, _NUM_INSTANCE, 4),
        ps[:, 4].reshape(_NUM_CLASSES, _NUM_INSTANCE),
        roi_mem,
        rs[:, :4].reshape(_NUM_CLASSES, _NUM_INSTANCE, 4),
        rs[:, 4].reshape(_NUM_CLASSES, _NUM_INSTANCE),
    )
